# baseline (speedup 1.0000x reference)
import functools

import jax
import jax.numpy as jnp
from jax import lax
from jax.experimental import pallas as pl
from jax.experimental.pallas import tpu as pltpu
from jax.experimental.pallas import tpu_sc as plsc

_B, _L, _H = 64, 40, 768
_NP = 576
_SEQ = _L + _NP + 1
_G, _P = 24, 16


def _make_sc_gather(num_ids, table_rows):
    info = plsc.get_sparse_core_info()
    nw = info.num_cores * info.num_subcores
    per_w = num_ids // nw
    mesh = plsc.VectorSubcoreMesh(core_axis_name="c", subcore_axis_name="s")

    @functools.partial(
        pl.kernel,
        mesh=mesh,
        out_type=jax.ShapeDtypeStruct((num_ids, _H), jnp.float32),
        scratch_types=[
            pltpu.VMEM((per_w,), jnp.int32),
            pltpu.VMEM((per_w, _H), jnp.float32),
            pltpu.SemaphoreType.DMA,
        ],
    )
    def gather(table_hbm, idx_hbm, out_hbm, idx_v, rows_v, sem):
        wid = lax.axis_index("s") * info.num_cores + lax.axis_index("c")
        base = wid * per_w
        pltpu.sync_copy(idx_hbm.at[pl.ds(base, per_w)], idx_v)
        pltpu.async_copy(table_hbm.at[idx_v], rows_v, sem).wait()
        pltpu.sync_copy(rows_v, out_hbm.at[pl.ds(base, per_w)])

    return gather


def _tc_body(text_ref, tt_ref, pos_t_ref, tte_ref, lng_ref, lnb_ref,
             patches_ref, w_ref, pb_ref, cls_ref, pos_i_ref, mod_ref,
             out_ref):
    t = tt_ref[0].astype(jnp.float32)
    onehot = jnp.concatenate([1.0 - t, t], axis=0)
    ttt = lax.dot_general(onehot, tte_ref[...],
                          (((0,), (0,)), ((), ())),
                          preferred_element_type=jnp.float32)
    x = text_ref[0] + pos_t_ref[...] + ttt
    m = jnp.mean(x, axis=-1, keepdims=True)
    v = jnp.mean((x - m) ** 2, axis=-1, keepdims=True)
    y = (x - m) / jnp.sqrt(v + 1e-12) * lng_ref[...] + lnb_ref[...]
    out_ref[0, :_L, :] = y + mod_ref[0:1, :]
    out_ref[0, _L:_L + 1, :] = cls_ref[0] + pos_i_ref[0, 0:1, :] + mod_ref[1:2, :]
    yi = jnp.dot(patches_ref[0].astype(jnp.bfloat16), w_ref[...],
                 preferred_element_type=jnp.float32)
    out_ref[0, _L + 1:, :] = yi + pb_ref[...] + pos_i_ref[0, 1:, :] + mod_ref[1:2, :]


def kernel(input_ids, attention_mask, token_type_ids, pixel_values, pixel_mask,
           word_emb, pos_emb_text, tok_type_emb, ln_g, ln_b,
           patch_w, patch_b, cls_token, pos_emb_img, modality_emb):
    ids = input_ids.reshape(-1).astype(jnp.int32)
    rows = _make_sc_gather(ids.shape[0], word_emb.shape[0])(word_emb, ids)
    text_raw = rows.reshape(_B, _L, _H)

    patch_w_bf = patch_w.astype(jnp.bfloat16)
    patches = (pixel_values
               .reshape(_B, 3, _G, _P, _G, _P)
               .transpose(0, 2, 4, 1, 3, 5)
               .reshape(_B, _NP, 3 * _P * _P))
    tt3 = token_type_ids.reshape(_B, 1, _L).astype(jnp.int32)

    emb = pl.pallas_call(
        _tc_body,
        grid=(_B,),
        in_specs=[
            pl.BlockSpec((1, _L, _H), lambda b: (b, 0, 0)),
            pl.BlockSpec((1, 1, _L), lambda b: (b, 0, 0)),
            pl.BlockSpec((_L, _H), lambda b: (0, 0)),
            pl.BlockSpec((2, _H), lambda b: (0, 0)),
            pl.BlockSpec((1, _H), lambda b: (0, 0)),
            pl.BlockSpec((1, _H), lambda b: (0, 0)),
            pl.BlockSpec((1, _NP, _H), lambda b: (b, 0, 0)),
            pl.BlockSpec((_H, _H), lambda b: (0, 0)),
            pl.BlockSpec((1, _H), lambda b: (0, 0)),
            pl.BlockSpec((1, 1, _H), lambda b: (0, 0, 0)),
            pl.BlockSpec((1, _NP + 1, _H), lambda b: (0, 0, 0)),
            pl.BlockSpec((2, _H), lambda b: (0, 0)),
        ],
        out_specs=pl.BlockSpec((1, _SEQ, _H), lambda b: (b, 0, 0)),
        out_shape=jax.ShapeDtypeStruct((_B, _SEQ, _H), jnp.float32),
    )(text_raw, tt3, pos_emb_text, tok_type_emb,
      ln_g.reshape(1, _H), ln_b.reshape(1, _H),
      patches, patch_w_bf, patch_b.reshape(1, _H),
      cls_token, pos_emb_img, modality_emb)

    img_mask = jnp.ones((_B, _NP + 1), dtype=attention_mask.dtype)
    masks = jnp.concatenate([attention_mask, img_mask], axis=1)
    return (emb, masks)


# speedup vs baseline: 1.1226x; 1.1226x over previous
import functools

import jax
import jax.numpy as jnp
from jax import lax
from jax.experimental import pallas as pl
from jax.experimental.pallas import tpu as pltpu
from jax.experimental.pallas import tpu_sc as plsc

_B, _L, _H = 64, 40, 768
_NP = 576
_SEQ = _L + _NP + 1
_G, _P = 24, 16


def _make_sc_gather(num_ids, table_rows):
    info = plsc.get_sparse_core_info()
    nw = info.num_cores * info.num_subcores
    per_w = num_ids // nw
    mesh = plsc.VectorSubcoreMesh(core_axis_name="c", subcore_axis_name="s")

    @functools.partial(
        pl.kernel,
        mesh=mesh,
        out_type=jax.ShapeDtypeStruct((num_ids, _H), jnp.float32),
        scratch_types=[
            pltpu.VMEM((per_w,), jnp.int32),
            pltpu.VMEM((per_w, _H), jnp.float32),
            pltpu.SemaphoreType.DMA,
        ],
    )
    def gather(table_hbm, idx_hbm, out_hbm, idx_v, rows_v, sem):
        wid = lax.axis_index("s") * info.num_cores + lax.axis_index("c")
        base = wid * per_w
        pltpu.sync_copy(idx_hbm.at[pl.ds(base, per_w)], idx_v)
        pltpu.async_copy(table_hbm.at[idx_v], rows_v, sem).wait()
        pltpu.sync_copy(rows_v, out_hbm.at[pl.ds(base, per_w)])

    return gather


def _tc_body(text_ref, tt_ref, pos_t_ref, tte_ref, lng_ref, lnb_ref,
             patches_ref, w_ref, pb_ref, cls_ref, pos_i_ref, mod_ref,
             out_ref):
    t = tt_ref[0].astype(jnp.float32)
    onehot = jnp.concatenate([1.0 - t, t], axis=0)
    ttt = lax.dot_general(onehot, tte_ref[...],
                          (((0,), (0,)), ((), ())),
                          preferred_element_type=jnp.float32)
    x = text_ref[0] + pos_t_ref[...] + ttt
    m = jnp.mean(x, axis=-1, keepdims=True)
    v = jnp.mean((x - m) ** 2, axis=-1, keepdims=True)
    y = (x - m) / jnp.sqrt(v + 1e-12) * lng_ref[...] + lnb_ref[...]
    out_ref[0, :_L, :] = y + mod_ref[0:1, :]
    out_ref[0, _L:_L + 1, :] = cls_ref[0] + pos_i_ref[0, 0:1, :] + mod_ref[1:2, :]
    yi = jnp.dot(patches_ref[0].astype(jnp.bfloat16), w_ref[...],
                 preferred_element_type=jnp.float32)
    out_ref[0, _L + 1:, :] = yi + pb_ref[...] + pos_i_ref[0, 1:, :] + mod_ref[1:2, :]


def kernel(input_ids, attention_mask, token_type_ids, pixel_values, pixel_mask,
           word_emb, pos_emb_text, tok_type_emb, ln_g, ln_b,
           patch_w, patch_b, cls_token, pos_emb_img, modality_emb):
    ids = input_ids.reshape(-1).astype(jnp.int32)
    rows = _make_sc_gather(ids.shape[0], word_emb.shape[0])(word_emb, ids)
    text_raw = rows.reshape(_B, _L, _H)

    patch_w_bf = patch_w.astype(jnp.bfloat16)
    patches = (pixel_values.astype(jnp.bfloat16)
               .reshape(_B, 3, _G, _P, _G, _P)
               .transpose(0, 2, 4, 1, 3, 5)
               .reshape(_B, _NP, 3 * _P * _P))
    tt3 = token_type_ids.reshape(_B, 1, _L).astype(jnp.int32)

    emb = pl.pallas_call(
        _tc_body,
        grid=(_B,),
        in_specs=[
            pl.BlockSpec((1, _L, _H), lambda b: (b, 0, 0)),
            pl.BlockSpec((1, 1, _L), lambda b: (b, 0, 0)),
            pl.BlockSpec((_L, _H), lambda b: (0, 0)),
            pl.BlockSpec((2, _H), lambda b: (0, 0)),
            pl.BlockSpec((1, _H), lambda b: (0, 0)),
            pl.BlockSpec((1, _H), lambda b: (0, 0)),
            pl.BlockSpec((1, _NP, _H), lambda b: (b, 0, 0)),
            pl.BlockSpec((_H, _H), lambda b: (0, 0)),
            pl.BlockSpec((1, _H), lambda b: (0, 0)),
            pl.BlockSpec((1, 1, _H), lambda b: (0, 0, 0)),
            pl.BlockSpec((1, _NP + 1, _H), lambda b: (0, 0, 0)),
            pl.BlockSpec((2, _H), lambda b: (0, 0)),
        ],
        out_specs=pl.BlockSpec((1, _SEQ, _H), lambda b: (b, 0, 0)),
        out_shape=jax.ShapeDtypeStruct((_B, _SEQ, _H), jnp.float32),
    )(text_raw, tt3, pos_emb_text, tok_type_emb,
      ln_g.reshape(1, _H), ln_b.reshape(1, _H),
      patches, patch_w_bf, patch_b.reshape(1, _H),
      cls_token, pos_emb_img, modality_emb)

    img_mask = jnp.ones((_B, _NP + 1), dtype=attention_mask.dtype)
    masks = jnp.concatenate([attention_mask, img_mask], axis=1)
    return (emb, masks)


# speedup vs baseline: 1.1933x; 1.0630x over previous
import functools

import jax
import jax.numpy as jnp
from jax import lax
from jax.experimental import pallas as pl
from jax.experimental.pallas import tpu as pltpu
from jax.experimental.pallas import tpu_sc as plsc

_B, _L, _H = 64, 40, 768
_NP = 576
_SEQ = _L + _NP + 1
_G, _P = 24, 16


def _make_sc_gather(num_ids, table_rows):
    info = plsc.get_sparse_core_info()
    nw = info.num_cores * info.num_subcores
    per_w = num_ids // nw
    mesh = plsc.VectorSubcoreMesh(core_axis_name="c", subcore_axis_name="s")

    @functools.partial(
        pl.kernel,
        mesh=mesh,
        out_type=jax.ShapeDtypeStruct((num_ids, _H), jnp.float32),
        scratch_types=[
            pltpu.VMEM((per_w,), jnp.int32),
            pltpu.VMEM((per_w, _H), jnp.float32),
            pltpu.SemaphoreType.DMA,
        ],
    )
    def gather(table_hbm, idx_hbm, out_hbm, idx_v, rows_v, sem):
        wid = lax.axis_index("s") * info.num_cores + lax.axis_index("c")
        base = wid * per_w
        pltpu.sync_copy(idx_hbm.at[pl.ds(base, per_w)], idx_v)
        pltpu.async_copy(table_hbm.at[idx_v], rows_v, sem).wait()
        pltpu.sync_copy(rows_v, out_hbm.at[pl.ds(base, per_w)])

    return gather


_BB = 4


def _tc_body(text_ref, tt_ref, pos_t_ref, tte_ref, lng_ref, lnb_ref,
             patches_ref, w_ref, pb_ref, cls_ref, pos_i_ref, mod_ref,
             out_ref):
    yi = jnp.dot(patches_ref[...].reshape(_BB * _NP, _H).astype(jnp.bfloat16),
                 w_ref[...], preferred_element_type=jnp.float32)
    for i in range(_BB):
        t = tt_ref[i].astype(jnp.float32)
        onehot = jnp.concatenate([1.0 - t, t], axis=0)
        ttt = lax.dot_general(onehot, tte_ref[...],
                              (((0,), (0,)), ((), ())),
                              preferred_element_type=jnp.float32)
        x = text_ref[i] + pos_t_ref[...] + ttt
        m = jnp.mean(x, axis=-1, keepdims=True)
        v = jnp.mean((x - m) ** 2, axis=-1, keepdims=True)
        y = (x - m) / jnp.sqrt(v + 1e-12) * lng_ref[...] + lnb_ref[...]
        out_ref[i, :_L, :] = y + mod_ref[0:1, :]
        out_ref[i, _L:_L + 1, :] = (cls_ref[0] + pos_i_ref[0, 0:1, :]
                                    + mod_ref[1:2, :])
        out_ref[i, _L + 1:, :] = (yi[i * _NP:(i + 1) * _NP, :] + pb_ref[...]
                                  + pos_i_ref[0, 1:, :] + mod_ref[1:2, :])


def kernel(input_ids, attention_mask, token_type_ids, pixel_values, pixel_mask,
           word_emb, pos_emb_text, tok_type_emb, ln_g, ln_b,
           patch_w, patch_b, cls_token, pos_emb_img, modality_emb):
    ids = input_ids.reshape(-1).astype(jnp.int32)
    rows = _make_sc_gather(ids.shape[0], word_emb.shape[0])(word_emb, ids)
    text_raw = rows.reshape(_B, _L, _H)

    patch_w_bf = patch_w.astype(jnp.bfloat16)
    patches = (pixel_values.astype(jnp.bfloat16)
               .reshape(_B, 3, _G, _P, _G, _P)
               .transpose(0, 2, 4, 1, 3, 5)
               .reshape(_B, _NP, 3 * _P * _P))
    tt3 = token_type_ids.reshape(_B, 1, _L).astype(jnp.int32)

    emb = pl.pallas_call(
        _tc_body,
        grid=(_B // _BB,),
        in_specs=[
            pl.BlockSpec((_BB, _L, _H), lambda b: (b, 0, 0)),
            pl.BlockSpec((_BB, 1, _L), lambda b: (b, 0, 0)),
            pl.BlockSpec((_L, _H), lambda b: (0, 0)),
            pl.BlockSpec((2, _H), lambda b: (0, 0)),
            pl.BlockSpec((1, _H), lambda b: (0, 0)),
            pl.BlockSpec((1, _H), lambda b: (0, 0)),
            pl.BlockSpec((_BB, _NP, _H), lambda b: (b, 0, 0)),
            pl.BlockSpec((_H, _H), lambda b: (0, 0)),
            pl.BlockSpec((1, _H), lambda b: (0, 0)),
            pl.BlockSpec((1, 1, _H), lambda b: (0, 0, 0)),
            pl.BlockSpec((1, _NP + 1, _H), lambda b: (0, 0, 0)),
            pl.BlockSpec((2, _H), lambda b: (0, 0)),
        ],
        out_specs=pl.BlockSpec((_BB, _SEQ, _H), lambda b: (b, 0, 0)),
        out_shape=jax.ShapeDtypeStruct((_B, _SEQ, _H), jnp.float32),
    )(text_raw, tt3, pos_emb_text, tok_type_emb,
      ln_g.reshape(1, _H), ln_b.reshape(1, _H),
      patches, patch_w_bf, patch_b.reshape(1, _H),
      cls_token, pos_emb_img, modality_emb)

    img_mask = jnp.ones((_B, _NP + 1), dtype=attention_mask.dtype)
    masks = jnp.concatenate([attention_mask, img_mask], axis=1)
    return (emb, masks)


# speedup vs baseline: 1.1987x; 1.0045x over previous
import functools

import jax
import jax.numpy as jnp
from jax import lax
from jax.experimental import pallas as pl
from jax.experimental.pallas import tpu as pltpu
from jax.experimental.pallas import tpu_sc as plsc

_B, _L, _H = 64, 40, 768
_NP = 576
_SEQ = _L + _NP + 1
_G, _P = 24, 16


def _make_sc_gather(num_ids, table_rows):
    info = plsc.get_sparse_core_info()
    nw = info.num_cores * info.num_subcores
    per_w = num_ids // nw
    mesh = plsc.VectorSubcoreMesh(core_axis_name="c", subcore_axis_name="s")

    @functools.partial(
        pl.kernel,
        mesh=mesh,
        out_type=jax.ShapeDtypeStruct((num_ids, _H), jnp.float32),
        scratch_types=[
            pltpu.VMEM((per_w,), jnp.int32),
            pltpu.VMEM((per_w, _H), jnp.float32),
            pltpu.SemaphoreType.DMA,
        ],
    )
    def gather(table_hbm, idx_hbm, out_hbm, idx_v, rows_v, sem):
        wid = lax.axis_index("s") * info.num_cores + lax.axis_index("c")
        base = wid * per_w
        pltpu.sync_copy(idx_hbm.at[pl.ds(base, per_w)], idx_v)
        pltpu.async_copy(table_hbm.at[idx_v], rows_v, sem).wait()
        pltpu.sync_copy(rows_v, out_hbm.at[pl.ds(base, per_w)])

    return gather


_BB = 8


def _tc_body(text_ref, tt_ref, pos_t_ref, tte_ref, lng_ref, lnb_ref,
             patches_ref, w_ref, pb_ref, cls_ref, pos_i_ref, mod_ref,
             out_ref):
    yi = jnp.dot(patches_ref[...].reshape(_BB * _NP, _H).astype(jnp.bfloat16),
                 w_ref[...], preferred_element_type=jnp.float32)
    for i in range(_BB):
        t = tt_ref[i].astype(jnp.float32)
        onehot = jnp.concatenate([1.0 - t, t], axis=0)
        ttt = lax.dot_general(onehot, tte_ref[...],
                              (((0,), (0,)), ((), ())),
                              preferred_element_type=jnp.float32)
        x = text_ref[i] + pos_t_ref[...] + ttt
        m = jnp.mean(x, axis=-1, keepdims=True)
        v = jnp.mean((x - m) ** 2, axis=-1, keepdims=True)
        y = (x - m) / jnp.sqrt(v + 1e-12) * lng_ref[...] + lnb_ref[...]
        out_ref[i, :_L, :] = y + mod_ref[0:1, :]
        out_ref[i, _L:_L + 1, :] = (cls_ref[0] + pos_i_ref[0, 0:1, :]
                                    + mod_ref[1:2, :])
        out_ref[i, _L + 1:, :] = (yi[i * _NP:(i + 1) * _NP, :] + pb_ref[...]
                                  + pos_i_ref[0, 1:, :] + mod_ref[1:2, :])


def kernel(input_ids, attention_mask, token_type_ids, pixel_values, pixel_mask,
           word_emb, pos_emb_text, tok_type_emb, ln_g, ln_b,
           patch_w, patch_b, cls_token, pos_emb_img, modality_emb):
    ids = input_ids.reshape(-1).astype(jnp.int32)
    rows = _make_sc_gather(ids.shape[0], word_emb.shape[0])(word_emb, ids)
    text_raw = rows.reshape(_B, _L, _H)

    patch_w_bf = patch_w.astype(jnp.bfloat16)
    patches = (pixel_values.astype(jnp.bfloat16)
               .reshape(_B, 3, _G, _P, _G, _P)
               .transpose(0, 2, 4, 1, 3, 5)
               .reshape(_B, _NP, 3 * _P * _P))
    tt3 = token_type_ids.reshape(_B, 1, _L).astype(jnp.int32)

    emb = pl.pallas_call(
        _tc_body,
        grid=(_B // _BB,),
        in_specs=[
            pl.BlockSpec((_BB, _L, _H), lambda b: (b, 0, 0)),
            pl.BlockSpec((_BB, 1, _L), lambda b: (b, 0, 0)),
            pl.BlockSpec((_L, _H), lambda b: (0, 0)),
            pl.BlockSpec((2, _H), lambda b: (0, 0)),
            pl.BlockSpec((1, _H), lambda b: (0, 0)),
            pl.BlockSpec((1, _H), lambda b: (0, 0)),
            pl.BlockSpec((_BB, _NP, _H), lambda b: (b, 0, 0)),
            pl.BlockSpec((_H, _H), lambda b: (0, 0)),
            pl.BlockSpec((1, _H), lambda b: (0, 0)),
            pl.BlockSpec((1, 1, _H), lambda b: (0, 0, 0)),
            pl.BlockSpec((1, _NP + 1, _H), lambda b: (0, 0, 0)),
            pl.BlockSpec((2, _H), lambda b: (0, 0)),
        ],
        out_specs=pl.BlockSpec((_BB, _SEQ, _H), lambda b: (b, 0, 0)),
        out_shape=jax.ShapeDtypeStruct((_B, _SEQ, _H), jnp.float32),
    )(text_raw, tt3, pos_emb_text, tok_type_emb,
      ln_g.reshape(1, _H), ln_b.reshape(1, _H),
      patches, patch_w_bf, patch_b.reshape(1, _H),
      cls_token, pos_emb_img, modality_emb)

    img_mask = jnp.ones((_B, _NP + 1), dtype=attention_mask.dtype)
    masks = jnp.concatenate([attention_mask, img_mask], axis=1)
    return (emb, masks)
